# Initial kernel scaffold; baseline (speedup 1.0000x reference)
import jax
import jax.numpy as jnp
from jax.experimental import pallas as pl


def kernel(h, edge_index, pos, batch, inv_W1, inv_b1, inv_W2, inv_b2, mod_W, mod_b, lin1_W, bn1_gamma, dir1, lin2_W, bn2_gamma, dir2):
    raise NotImplementedError("write your pallas kernel here")


# speedup vs baseline: 15.8058x; 15.8058x over previous
import functools
from functools import partial

import jax
import jax.numpy as jnp
from jax import lax
from jax.experimental import pallas as pl
from jax.experimental.pallas import tpu as pltpu
from jax.experimental.pallas import tpu_sc as plsc

EPS_BN = 1e-5
SLOPE = 0.2

CH = 128
NC = 2
NS = 16
NW = NC * NS
BLK = 4096


def _sc_gather4(h48, pos8, src, tgt, EP):
    N = h48.shape[0]
    CPW = EP // (CH * NW)
    mesh = plsc.VectorSubcoreMesh(core_axis_name="c", subcore_axis_name="s")

    @partial(
        pl.kernel, mesh=mesh,
        compiler_params=pltpu.CompilerParams(use_tc_tiling_on_sc=False),
        out_type=[
            jax.ShapeDtypeStruct((EP, 48), jnp.float32),
            jax.ShapeDtypeStruct((EP, 48), jnp.float32),
            jax.ShapeDtypeStruct((EP, 8), jnp.float32),
            jax.ShapeDtypeStruct((EP, 8), jnp.float32),
        ],
        scratch_types=[
            pltpu.VMEM((CH,), jnp.int32),
            pltpu.VMEM((CH,), jnp.int32),
            pltpu.VMEM((CH, 48), jnp.float32),
            pltpu.VMEM((CH, 48), jnp.float32),
            pltpu.VMEM((CH, 8), jnp.float32),
            pltpu.VMEM((CH, 8), jnp.float32),
            pltpu.SemaphoreType.DMA,
            pltpu.SemaphoreType.DMA,
            pltpu.SemaphoreType.DMA,
            pltpu.SemaphoreType.DMA,
        ],
    )
    def k(h_hbm, p_hbm, src_hbm, tgt_hbm, hs_out, ht_out, ps_out, pt_out,
          si_v, ti_v, hs_v, ht_v, ps_v, pt_v, sem0, sem1, sem2, sem3):
        w = lax.axis_index("s") * NC + lax.axis_index("c")

        def body(kk, _):
            off = (w * CPW + kk) * CH
            pltpu.sync_copy(src_hbm.at[pl.ds(off, CH)], si_v)
            pltpu.sync_copy(tgt_hbm.at[pl.ds(off, CH)], ti_v)
            c0 = pltpu.async_copy(h_hbm.at[si_v], hs_v, sem0)
            c1 = pltpu.async_copy(h_hbm.at[ti_v], ht_v, sem1)
            c2 = pltpu.async_copy(p_hbm.at[si_v], ps_v, sem2)
            c3 = pltpu.async_copy(p_hbm.at[ti_v], pt_v, sem3)
            c0.wait(); c1.wait(); c2.wait(); c3.wait()
            pltpu.sync_copy(hs_v, hs_out.at[pl.ds(off, CH), :])
            pltpu.sync_copy(ht_v, ht_out.at[pl.ds(off, CH), :])
            pltpu.sync_copy(ps_v, ps_out.at[pl.ds(off, CH), :])
            pltpu.sync_copy(pt_v, pt_out.at[pl.ds(off, CH), :])
            return _

        lax.fori_loop(0, CPW, body, None)

    return k(h48, pos8, src, tgt)


def _sc_gather1(tab, src, EP):
    CPW = EP // (CH * NW)
    mesh = plsc.VectorSubcoreMesh(core_axis_name="c", subcore_axis_name="s")

    @partial(
        pl.kernel, mesh=mesh,
        compiler_params=pltpu.CompilerParams(use_tc_tiling_on_sc=False),
        out_type=jax.ShapeDtypeStruct((EP, 8), jnp.float32),
        scratch_types=[
            pltpu.VMEM((CH,), jnp.int32),
            pltpu.VMEM((CH, 8), jnp.float32),
            pltpu.SemaphoreType.DMA,
        ],
    )
    def k(tab_hbm, src_hbm, out_hbm, si_v, v_v, sem):
        w = lax.axis_index("s") * NC + lax.axis_index("c")

        def body(kk, _):
            off = (w * CPW + kk) * CH
            pltpu.sync_copy(src_hbm.at[pl.ds(off, CH)], si_v)
            pltpu.async_copy(tab_hbm.at[si_v], v_v, sem).wait()
            pltpu.sync_copy(v_v, out_hbm.at[pl.ds(off, CH), :])
            return _

        lax.fori_loop(0, CPW, body, None)

    return k(tab, src)


def _sc_scatter_mean1(src_m, vals, EP, NHALF, SR, NACC, z1d):
    NH_OUT = 16 * SR
    GARB = NH_OUT
    CPS = EP // (NS * CH)
    mesh = plsc.VectorSubcoreMesh(core_axis_name="c", subcore_axis_name="s")

    @partial(
        pl.kernel, mesh=mesh,
        compiler_params=pltpu.CompilerParams(use_tc_tiling_on_sc=False),
        out_type=jax.ShapeDtypeStruct((NC, NH_OUT), jnp.float32),
        scratch_types=[
            pltpu.VMEM((CH,), jnp.int32),
            pltpu.VMEM((CH,), jnp.int32),
            pltpu.VMEM((CH,), jnp.float32),
            pltpu.VMEM((CH,), jnp.float32),
            pltpu.VMEM((NACC // NS,), jnp.float32),
            pltpu.VMEM((SR,), jnp.float32),
            pltpu.VMEM((SR,), jnp.float32),
            pltpu.VMEM((SR,), jnp.float32),
            pltpu.VMEM_SHARED((NACC,), jnp.float32),
            pltpu.VMEM_SHARED((NACC,), jnp.float32),
        ],
    )
    def k(src_hbm, val_hbm, z_hbm, out_hbm,
          si_v, ix_v, vv_v, one_v, zz_v, sb_v, cb_v, mb_v, sum_sh, cnt_sh):
        c = lax.axis_index("c")
        s = lax.axis_index("s")
        zs = NACC // NS

        pltpu.sync_copy(z_hbm, zz_v)
        pltpu.sync_copy(zz_v, sum_sh.at[pl.ds(s * zs, zs)])
        pltpu.sync_copy(zz_v, cnt_sh.at[pl.ds(s * zs, zs)])
        for j in range(CH // 16):
            one_v[pl.ds(j * 16, 16)] = jnp.ones((16,), jnp.float32)
        plsc.subcore_barrier()

        base = jnp.int32(c * NHALF)

        def body(kk, _):
            off = (s * CPS + kk) * CH
            pltpu.sync_copy(src_hbm.at[pl.ds(off, CH)], si_v)
            pltpu.sync_copy(val_hbm.at[pl.ds(off, CH)], vv_v)
            for j in range(CH // 16):
                iv = si_v[pl.ds(j * 16, 16)] - base
                ok = (iv >= 0) & (iv < NHALF)
                ix_v[pl.ds(j * 16, 16)] = jnp.where(ok, iv, GARB)
            pltpu.sync_copy(vv_v, sum_sh.at[ix_v], add=True)
            pltpu.sync_copy(one_v, cnt_sh.at[ix_v], add=True)
            return _

        lax.fori_loop(0, CPS, body, None)
        plsc.subcore_barrier()

        pltpu.sync_copy(sum_sh.at[pl.ds(s * SR, SR)], sb_v)
        pltpu.sync_copy(cnt_sh.at[pl.ds(s * SR, SR)], cb_v)
        for j in range(SR // 16):
            sl = pl.ds(j * 16, 16)
            mb_v[sl] = sb_v[sl] / jnp.maximum(cb_v[sl], 1.0)
        pltpu.sync_copy(mb_v, out_hbm.at[c, pl.ds(s * SR, SR)])

    return k(src_m, vals, z1d)


def _sc_scatter_rows(tgt_m, rows, EP, NHALF, SR, NACC, z1d, z2d):
    NH_OUT = 16 * SR
    GARB = NH_OUT
    CPS = EP // (NS * CH)
    ZR = 64
    mesh = plsc.VectorSubcoreMesh(core_axis_name="c", subcore_axis_name="s")

    @partial(
        pl.kernel, mesh=mesh,
        compiler_params=pltpu.CompilerParams(use_tc_tiling_on_sc=False),
        out_type=[
            jax.ShapeDtypeStruct((NC, NH_OUT, 48), jnp.float32),
            jax.ShapeDtypeStruct((NC, NH_OUT), jnp.float32),
        ],
        scratch_types=[
            pltpu.VMEM((CH,), jnp.int32),
            pltpu.VMEM((CH,), jnp.int32),
            pltpu.VMEM((CH, 48), jnp.float32),
            pltpu.VMEM((CH,), jnp.float32),
            pltpu.VMEM((NACC // NS,), jnp.float32),
            pltpu.VMEM((ZR, 48), jnp.float32),
            pltpu.VMEM((56, 48), jnp.float32),
            pltpu.VMEM((SR,), jnp.float32),
            pltpu.VMEM_SHARED((NACC, 48), jnp.float32),
            pltpu.VMEM_SHARED((NACC,), jnp.float32),
        ],
    )
    def k(tgt_hbm, rows_hbm, z1_hbm, z2_hbm, sum_out, cnt_out,
          ti_v, ix_v, rv_v, one_v, zz_v, z2_v, rb_v, cb_v, acc_sh, cnt_sh):
        c = lax.axis_index("c")
        s = lax.axis_index("s")
        zs = NACC // NS

        pltpu.sync_copy(z1_hbm, zz_v)
        pltpu.sync_copy(z2_hbm, z2_v)
        pltpu.sync_copy(zz_v, cnt_sh.at[pl.ds(s * zs, zs)])

        def zbody(kk, _):
            r0 = s * zs + kk * ZR
            pltpu.sync_copy(z2_v, acc_sh.at[pl.ds(r0, ZR), :])
            return _

        lax.fori_loop(0, zs // ZR, zbody, None)
        for j in range(CH // 16):
            one_v[pl.ds(j * 16, 16)] = jnp.ones((16,), jnp.float32)
        plsc.subcore_barrier()

        base = jnp.int32(c * NHALF)

        def body(kk, _):
            off = (s * CPS + kk) * CH
            pltpu.sync_copy(tgt_hbm.at[pl.ds(off, CH)], ti_v)
            pltpu.sync_copy(rows_hbm.at[pl.ds(off, CH), :], rv_v)
            for j in range(CH // 16):
                iv = ti_v[pl.ds(j * 16, 16)] - base
                ok = (iv >= 0) & (iv < NHALF)
                ix_v[pl.ds(j * 16, 16)] = jnp.where(ok, iv, GARB)
            pltpu.sync_copy(rv_v, acc_sh.at[ix_v], add=True)
            pltpu.sync_copy(one_v, cnt_sh.at[ix_v], add=True)
            return _

        lax.fori_loop(0, CPS, body, None)
        plsc.subcore_barrier()

        pltpu.sync_copy(cnt_sh.at[pl.ds(s * SR, SR)], cb_v)
        pltpu.sync_copy(cb_v, cnt_out.at[c, pl.ds(s * SR, SR)])

        def wbody(kk, _):
            r0 = s * SR + kk * 56
            pltpu.sync_copy(acc_sh.at[pl.ds(r0, 56), :], rb_v)
            pltpu.sync_copy(rb_v, sum_out.at[c, pl.ds(r0, 56), :])
            return _

        lax.fori_loop(0, SR // 56, wbody, None)

    return k(tgt_m, rows, z1d, z2d)


def _mm(a, b):
    return jax.lax.dot_general(a, b, (((1,), (0,)), ((), ())),
                               preferred_element_type=jnp.float32,
                               precision=jax.lax.Precision.HIGHEST)


def _mmb(a, b):
    return jax.lax.dot_general(a.astype(jnp.bfloat16), b.astype(jnp.bfloat16),
                               (((1,), (0,)), ((), ())),
                               preferred_element_type=jnp.float32)


def _b32(x):
    return x.astype(jnp.bfloat16).astype(jnp.float32)


def _tc_pass1(hs, ht, ps, pt, B1a, B1b, B1c, S, E):
    EP = hs.shape[0]
    grid = EP // BLK

    def body(hs_r, ht_r, ps_r, pt_r, b1a_r, b1b_r, b1c_r, s_r,
             x1_r, dist_r, st_r):
        i = pl.program_id(0)
        r8 = pt_r[...] - ps_r[...]
        lane = lax.broadcasted_iota(jnp.int32, (BLK, 8), 1)
        r8 = jnp.where(lane < 3, r8, 0.0)
        d2 = jnp.sum(r8 * r8, axis=1, keepdims=True)
        dist = jnp.sqrt(d2)
        dist_r[...] = dist
        x1 = (_mmb(hs_r[...], b1a_r[...]) + _mmb(ht_r[...], b1b_r[...])
              + _mmb(r8, b1c_r[...]))
        x1_r[...] = x1
        n1 = jnp.sqrt(_mm(x1 * x1, s_r[...]))
        row = i * BLK + lax.broadcasted_iota(jnp.int32, (BLK, 16), 0)
        n1 = jnp.where(row < E, n1, 0.0)
        st = jnp.stack([jnp.sum(n1, axis=0), jnp.sum(n1 * n1, axis=0)])

        @pl.when(i == 0)
        def _():
            st_r[...] = jnp.zeros_like(st_r)

        st_r[...] += st

    return pl.pallas_call(
        body,
        grid=(grid,),
        in_specs=[
            pl.BlockSpec((BLK, 48), lambda i: (i, 0)),
            pl.BlockSpec((BLK, 48), lambda i: (i, 0)),
            pl.BlockSpec((BLK, 8), lambda i: (i, 0)),
            pl.BlockSpec((BLK, 8), lambda i: (i, 0)),
            pl.BlockSpec((48, 48), lambda i: (0, 0)),
            pl.BlockSpec((48, 48), lambda i: (0, 0)),
            pl.BlockSpec((8, 48), lambda i: (0, 0)),
            pl.BlockSpec((48, 16), lambda i: (0, 0)),
        ],
        out_specs=[
            pl.BlockSpec((BLK, 48), lambda i: (i, 0)),
            pl.BlockSpec((BLK, 1), lambda i: (i, 0)),
            pl.BlockSpec((2, 16), lambda i: (0, 0)),
        ],
        out_shape=[
            jax.ShapeDtypeStruct((EP, 48), jnp.float32),
            jax.ShapeDtypeStruct((EP, 1), jnp.float32),
            jax.ShapeDtypeStruct((2, 16), jnp.float32),
        ],
    )(hs, ht, ps, pt, B1a, B1b, B1c, S)


def _bn_lrelu(x, S, ST, mean, std, gamma, dhat):
    nsq = _mm(x * x, S)
    n = jnp.sqrt(nsq)
    f = (n - mean) / ((n + EPS_BN) * std) * gamma
    xbn = x * _mm(f, ST)
    proj = _mm(xbn * dhat, S)
    fac = SLOPE + (1.0 - SLOPE) * (proj >= 0).astype(jnp.float32)
    return xbn * _mm(fac, ST)


def _tc_pass2(x1, B2, S, ST, mean1, std1, gamma1, dhat1, E):
    EP = x1.shape[0]
    grid = EP // BLK

    def body(x1_r, b2_r, s_r, st_rr, m_r, sd_r, g_r, d_r, x2_r, st_r):
        i = pl.program_id(0)
        y1 = _bn_lrelu(x1_r[...], s_r[...], st_rr[...],
                       m_r[...], sd_r[...], g_r[...], d_r[...])
        x2 = _mmb(y1, b2_r[...])
        x2_r[...] = x2
        n2 = jnp.sqrt(_mm(x2 * x2, s_r[...]))
        row = i * BLK + lax.broadcasted_iota(jnp.int32, (BLK, 16), 0)
        n2 = jnp.where(row < E, n2, 0.0)
        st = jnp.stack([jnp.sum(n2, axis=0), jnp.sum(n2 * n2, axis=0)])

        @pl.when(i == 0)
        def _():
            st_r[...] = jnp.zeros_like(st_r)

        st_r[...] += st

    return pl.pallas_call(
        body,
        grid=(grid,),
        in_specs=[
            pl.BlockSpec((BLK, 48), lambda i: (i, 0)),
            pl.BlockSpec((48, 48), lambda i: (0, 0)),
            pl.BlockSpec((48, 16), lambda i: (0, 0)),
            pl.BlockSpec((16, 48), lambda i: (0, 0)),
            pl.BlockSpec((1, 16), lambda i: (0, 0)),
            pl.BlockSpec((1, 16), lambda i: (0, 0)),
            pl.BlockSpec((1, 16), lambda i: (0, 0)),
            pl.BlockSpec((1, 48), lambda i: (0, 0)),
        ],
        out_specs=[
            pl.BlockSpec((BLK, 48), lambda i: (i, 0)),
            pl.BlockSpec((2, 16), lambda i: (0, 0)),
        ],
        out_shape=[
            jax.ShapeDtypeStruct((EP, 48), jnp.float32),
            jax.ShapeDtypeStruct((2, 16), jnp.float32),
        ],
    )(x1, B2, S, ST, mean1, std1, gamma1, dhat1)


def _tc_pass3(x2, dist, ld, S, ST, mean2, std2, gamma2, dhat2,
              W1T, b1, W2T, b2, WmT, bm):
    EP = x2.shape[0]
    grid = EP // BLK

    def body(x2_r, dist_r, ld_r, s_r, st_rr, m_r, sd_r, g_r, d_r,
             w1_r, b1_r, w2_r, b2_r, wm_r, bm_r, mf_r):
        y2 = _bn_lrelu(x2_r[...], s_r[...], st_rr[...],
                       m_r[...], sd_r[...], g_r[...], d_r[...])
        dist = dist_r[...]
        ld = ld_r[...]
        w1 = _b32(w1_r[...])
        zp = (_b32(dist) * w1[0:1, :] + _b32(dist / (ld + 1e-6)) * w1[1:2, :]
              + _b32(jnp.log(dist + 1e-6)) * w1[2:3, :] + _b32(ld) * w1[3:4, :]
              + b1_r[...])
        z = zp * jax.nn.sigmoid(zp)
        ie = _mmb(z, w2_r[...]) + b2_r[...]
        mod = jax.nn.sigmoid(_mmb(ie, wm_r[...]) + bm_r[...])
        mf_r[...] = y2 * _mm(mod, st_rr[...])

    return pl.pallas_call(
        body,
        grid=(grid,),
        in_specs=[
            pl.BlockSpec((BLK, 48), lambda i: (i, 0)),
            pl.BlockSpec((BLK, 1), lambda i: (i, 0)),
            pl.BlockSpec((BLK, 1), lambda i: (i, 0)),
            pl.BlockSpec((48, 16), lambda i: (0, 0)),
            pl.BlockSpec((16, 48), lambda i: (0, 0)),
            pl.BlockSpec((1, 16), lambda i: (0, 0)),
            pl.BlockSpec((1, 16), lambda i: (0, 0)),
            pl.BlockSpec((1, 16), lambda i: (0, 0)),
            pl.BlockSpec((1, 48), lambda i: (0, 0)),
            pl.BlockSpec((4, 16), lambda i: (0, 0)),
            pl.BlockSpec((1, 16), lambda i: (0, 0)),
            pl.BlockSpec((16, 16), lambda i: (0, 0)),
            pl.BlockSpec((1, 16), lambda i: (0, 0)),
            pl.BlockSpec((16, 16), lambda i: (0, 0)),
            pl.BlockSpec((1, 16), lambda i: (0, 0)),
        ],
        out_specs=pl.BlockSpec((BLK, 48), lambda i: (i, 0)),
        out_shape=jax.ShapeDtypeStruct((EP, 48), jnp.float32),
    )(x2, dist, ld, S, ST, mean2, std2, gamma2, dhat2,
      W1T, b1, W2T, b2, WmT, bm)


def _tc_divide(sums, cnt):
    NP = sums.shape[0]
    blk = 1024
    grid = NP // blk

    def body(s_r, c_r, o_r):
        o_r[...] = s_r[...] / jnp.maximum(c_r[...], 1.0)

    return pl.pallas_call(
        body,
        grid=(grid,),
        in_specs=[
            pl.BlockSpec((blk, 48), lambda i: (i, 0)),
            pl.BlockSpec((blk, 1), lambda i: (i, 0)),
        ],
        out_specs=pl.BlockSpec((blk, 48), lambda i: (i, 0)),
        out_shape=jax.ShapeDtypeStruct((NP, 48), jnp.float32),
    )(sums, cnt)


def kernel(h, edge_index, pos, batch, inv_W1, inv_b1, inv_W2, inv_b2,
           mod_W, mod_b, lin1_W, bn1_gamma, dir1, lin2_W, bn2_gamma, dir2):
    N = h.shape[0]
    E = edge_index.shape[1]
    C = h.shape[1]

    EP = -(-E // (CH * NW)) * (CH * NW)
    NHALF = -(-N // 2)
    SR = -(-NHALF // (NS * 16)) * 16
    NH_OUT = NS * SR
    NACC = -(-(NH_OUT + 16) // (NS * 64)) * NS * 64

    src = edge_index[0].astype(jnp.int32)
    tgt = edge_index[1].astype(jnp.int32)
    pad = EP - E
    src0 = jnp.pad(src, (0, pad))
    tgt0 = jnp.pad(tgt, (0, pad))
    srcm = jnp.pad(src, (0, pad), constant_values=-1)
    tgtm = jnp.pad(tgt, (0, pad), constant_values=-1)
    h48 = h.reshape(N, 3 * C)
    pos8 = jnp.pad(pos, ((0, 0), (0, 5)))

    eye3 = jnp.eye(3, dtype=jnp.float32)
    B1a = jnp.kron(lin1_W[:, 0:C].T, eye3)
    B1b = jnp.kron(lin1_W[:, C:2 * C].T, eye3)
    B1c = jnp.kron(lin1_W[:, 2 * C:].T, eye3)
    B1c = jnp.pad(B1c, ((0, 8 - B1c.shape[0]), (0, 0)))
    B2 = jnp.kron(lin2_W.T, eye3)
    S = jnp.kron(jnp.eye(C, dtype=jnp.float32), jnp.ones((3, 1), jnp.float32))
    ST = S.T

    def _dhat(d):
        dd = d / jnp.clip(jnp.linalg.norm(d, axis=-1, keepdims=True),
                          1e-12, None)
        return dd.reshape(1, 3 * C)

    dhat1 = _dhat(dir1)
    dhat2 = _dhat(dir2)
    g1 = bn1_gamma.reshape(1, C)
    g2 = bn2_gamma.reshape(1, C)

    z1d = jnp.zeros((NACC // NS,), jnp.float32)
    z2d = jnp.zeros((64, 48), jnp.float32)

    hs, ht, ps, pt = _sc_gather4(h48, pos8, src0, tgt0, EP)
    x1, dist, st1 = _tc_pass1(hs, ht, ps, pt, B1a, B1b, B1c, S, E)

    nmd2 = _sc_scatter_mean1(srcm, dist.reshape(EP), EP, NHALF, SR, NACC, z1d)
    nmd = jnp.concatenate([nmd2[0, :NHALF], nmd2[1, :N - NHALF]])
    nmd8 = jnp.tile(nmd.reshape(N, 1), (1, 8))
    ld = _sc_gather1(nmd8, src0, EP)[:, :1]

    def _finalize(st):
        mean = st[0] / E
        var = (st[1] - E * mean * mean) / (E - 1)
        std = jnp.sqrt(jnp.maximum(var, 0.0)) + EPS_BN
        return mean.reshape(1, C), std.reshape(1, C)

    mean1, std1 = _finalize(st1)
    x2, st2 = _tc_pass2(x1, B2, S, ST, mean1, std1, g1, dhat1, E)
    mean2, std2 = _finalize(st2)

    mf = _tc_pass3(x2, dist, ld, S, ST, mean2, std2, g2, dhat2,
                   inv_W1.T, inv_b1.reshape(1, C), inv_W2.T,
                   inv_b2.reshape(1, C), mod_W.T, mod_b.reshape(1, C))

    msum, mcnt = _sc_scatter_rows(tgtm, mf, EP, NHALF, SR, NACC, z1d, z2d)
    sums = jnp.concatenate([msum[0, :NHALF], msum[1, :N - NHALF]], axis=0)
    cnts = jnp.concatenate([mcnt[0, :NHALF], mcnt[1, :N - NHALF]])
    NP6 = -(-N // 1024) * 1024
    sums = jnp.pad(sums, ((0, NP6 - N), (0, 0)))
    cnts = jnp.pad(cnts, (0, NP6 - N)).reshape(NP6, 1)
    out = _tc_divide(sums, cnts)[:N]
    return out.reshape(N, C, 3)


# speedup vs baseline: 25.2445x; 1.5972x over previous
import functools
from functools import partial

import jax
import jax.numpy as jnp
from jax import lax
from jax.experimental import pallas as pl
from jax.experimental.pallas import tpu as pltpu
from jax.experimental.pallas import tpu_sc as plsc

EPS_BN = 1e-5
SLOPE = 0.2

CH = 128
NC = 2
NS = 16
NW = NC * NS
BLK = 4096


def _sc_gather4(h48, pos8, src, tgt, EP):
    N = h48.shape[0]
    CPW = EP // (CH * NW)
    mesh = plsc.VectorSubcoreMesh(core_axis_name="c", subcore_axis_name="s")

    @partial(
        pl.kernel, mesh=mesh,
        compiler_params=pltpu.CompilerParams(use_tc_tiling_on_sc=False),
        out_type=[
            jax.ShapeDtypeStruct((EP, 48), jnp.float32),
            jax.ShapeDtypeStruct((EP, 48), jnp.float32),
            jax.ShapeDtypeStruct((EP, 8), jnp.float32),
            jax.ShapeDtypeStruct((EP, 8), jnp.float32),
        ],
        scratch_types=[
            pltpu.VMEM((CH,), jnp.int32),
            pltpu.VMEM((CH,), jnp.int32),
            pltpu.VMEM((CH, 48), jnp.float32),
            pltpu.VMEM((CH, 48), jnp.float32),
            pltpu.VMEM((CH, 8), jnp.float32),
            pltpu.VMEM((CH, 8), jnp.float32),
            pltpu.SemaphoreType.DMA,
            pltpu.SemaphoreType.DMA,
            pltpu.SemaphoreType.DMA,
            pltpu.SemaphoreType.DMA,
        ],
    )
    def k(h_hbm, p_hbm, src_hbm, tgt_hbm, hs_out, ht_out, ps_out, pt_out,
          si_v, ti_v, hs_v, ht_v, ps_v, pt_v, sem0, sem1, sem2, sem3):
        w = lax.axis_index("s") * NC + lax.axis_index("c")

        def body(kk, _):
            off = (w * CPW + kk) * CH
            pltpu.sync_copy(src_hbm.at[pl.ds(off, CH)], si_v)
            pltpu.sync_copy(tgt_hbm.at[pl.ds(off, CH)], ti_v)
            c0 = pltpu.async_copy(h_hbm.at[si_v], hs_v, sem0)
            c1 = pltpu.async_copy(h_hbm.at[ti_v], ht_v, sem1)
            c2 = pltpu.async_copy(p_hbm.at[si_v], ps_v, sem2)
            c3 = pltpu.async_copy(p_hbm.at[ti_v], pt_v, sem3)
            c0.wait(); c1.wait(); c2.wait(); c3.wait()
            pltpu.sync_copy(hs_v, hs_out.at[pl.ds(off, CH), :])
            pltpu.sync_copy(ht_v, ht_out.at[pl.ds(off, CH), :])
            pltpu.sync_copy(ps_v, ps_out.at[pl.ds(off, CH), :])
            pltpu.sync_copy(pt_v, pt_out.at[pl.ds(off, CH), :])
            return _

        lax.fori_loop(0, CPW, body, None)

    return k(h48, pos8, src, tgt)


def _sc_gather1(tab, src, EP):
    CPW = EP // (CH * NW)
    mesh = plsc.VectorSubcoreMesh(core_axis_name="c", subcore_axis_name="s")

    @partial(
        pl.kernel, mesh=mesh,
        compiler_params=pltpu.CompilerParams(use_tc_tiling_on_sc=False),
        out_type=jax.ShapeDtypeStruct((EP, 8), jnp.float32),
        scratch_types=[
            pltpu.VMEM((CH,), jnp.int32),
            pltpu.VMEM((CH, 8), jnp.float32),
            pltpu.SemaphoreType.DMA,
        ],
    )
    def k(tab_hbm, src_hbm, out_hbm, si_v, v_v, sem):
        w = lax.axis_index("s") * NC + lax.axis_index("c")

        def body(kk, _):
            off = (w * CPW + kk) * CH
            pltpu.sync_copy(src_hbm.at[pl.ds(off, CH)], si_v)
            pltpu.async_copy(tab_hbm.at[si_v], v_v, sem).wait()
            pltpu.sync_copy(v_v, out_hbm.at[pl.ds(off, CH), :])
            return _

        lax.fori_loop(0, CPW, body, None)

    return k(tab, src)


def _sc_scatter_mean1(src_m, vals, EP, NHALF, SR, NACC, z1d):
    NH_OUT = 16 * SR
    GARB = NH_OUT
    CPS = EP // (NS * CH)
    mesh = plsc.VectorSubcoreMesh(core_axis_name="c", subcore_axis_name="s")

    @partial(
        pl.kernel, mesh=mesh,
        compiler_params=pltpu.CompilerParams(use_tc_tiling_on_sc=False),
        out_type=jax.ShapeDtypeStruct((NC, NH_OUT), jnp.float32),
        scratch_types=[
            pltpu.VMEM((CH,), jnp.int32),
            pltpu.VMEM((CH,), jnp.int32),
            pltpu.VMEM((CH,), jnp.float32),
            pltpu.VMEM((CH,), jnp.float32),
            pltpu.VMEM((NACC // NS,), jnp.float32),
            pltpu.VMEM((SR,), jnp.float32),
            pltpu.VMEM((SR,), jnp.float32),
            pltpu.VMEM((SR,), jnp.float32),
            pltpu.VMEM_SHARED((NACC,), jnp.float32),
            pltpu.VMEM_SHARED((NACC,), jnp.float32),
        ],
    )
    def k(src_hbm, val_hbm, z_hbm, out_hbm,
          si_v, ix_v, vv_v, one_v, zz_v, sb_v, cb_v, mb_v, sum_sh, cnt_sh):
        c = lax.axis_index("c")
        s = lax.axis_index("s")
        zs = NACC // NS

        pltpu.sync_copy(z_hbm, zz_v)
        pltpu.sync_copy(zz_v, sum_sh.at[pl.ds(s * zs, zs)])
        pltpu.sync_copy(zz_v, cnt_sh.at[pl.ds(s * zs, zs)])
        for j in range(CH // 16):
            one_v[pl.ds(j * 16, 16)] = jnp.ones((16,), jnp.float32)
        plsc.subcore_barrier()

        base = jnp.int32(c * NHALF)

        def body(kk, _):
            off = (s * CPS + kk) * CH
            pltpu.sync_copy(src_hbm.at[pl.ds(off, CH)], si_v)
            pltpu.sync_copy(val_hbm.at[pl.ds(off, CH)], vv_v)
            for j in range(CH // 16):
                iv = si_v[pl.ds(j * 16, 16)] - base
                ok = (iv >= 0) & (iv < NHALF)
                ix_v[pl.ds(j * 16, 16)] = jnp.where(ok, iv, GARB)
            pltpu.sync_copy(vv_v, sum_sh.at[ix_v], add=True)
            pltpu.sync_copy(one_v, cnt_sh.at[ix_v], add=True)
            return _

        lax.fori_loop(0, CPS, body, None)
        plsc.subcore_barrier()

        pltpu.sync_copy(sum_sh.at[pl.ds(s * SR, SR)], sb_v)
        pltpu.sync_copy(cnt_sh.at[pl.ds(s * SR, SR)], cb_v)
        for j in range(SR // 16):
            sl = pl.ds(j * 16, 16)
            mb_v[sl] = sb_v[sl] / jnp.maximum(cb_v[sl], 1.0)
        pltpu.sync_copy(mb_v, out_hbm.at[c, pl.ds(s * SR, SR)])

    return k(src_m, vals, z1d)


def _sc_scatter_rows(tgt_m, rows, EP, NHALF, SR, NACC, z1d, z2d):
    NH_OUT = 16 * SR
    GARB = NH_OUT
    CPS = EP // (NS * CH)
    ZR = 64
    mesh = plsc.VectorSubcoreMesh(core_axis_name="c", subcore_axis_name="s")

    @partial(
        pl.kernel, mesh=mesh,
        compiler_params=pltpu.CompilerParams(use_tc_tiling_on_sc=False),
        out_type=[
            jax.ShapeDtypeStruct((NC, NH_OUT, 48), jnp.float32),
            jax.ShapeDtypeStruct((NC, NH_OUT), jnp.float32),
        ],
        scratch_types=[
            pltpu.VMEM((CH,), jnp.int32),
            pltpu.VMEM((CH,), jnp.int32),
            pltpu.VMEM((CH, 48), jnp.float32),
            pltpu.VMEM((CH,), jnp.float32),
            pltpu.VMEM((NACC // NS,), jnp.float32),
            pltpu.VMEM((ZR, 48), jnp.float32),
            pltpu.VMEM((56, 48), jnp.float32),
            pltpu.VMEM((SR,), jnp.float32),
            pltpu.VMEM_SHARED((NACC, 48), jnp.float32),
            pltpu.VMEM_SHARED((NACC,), jnp.float32),
        ],
    )
    def k(tgt_hbm, rows_hbm, z1_hbm, z2_hbm, sum_out, cnt_out,
          ti_v, ix_v, rv_v, one_v, zz_v, z2_v, rb_v, cb_v, acc_sh, cnt_sh):
        c = lax.axis_index("c")
        s = lax.axis_index("s")
        zs = NACC // NS

        pltpu.sync_copy(z1_hbm, zz_v)
        pltpu.sync_copy(z2_hbm, z2_v)
        pltpu.sync_copy(zz_v, cnt_sh.at[pl.ds(s * zs, zs)])

        def zbody(kk, _):
            r0 = s * zs + kk * ZR
            pltpu.sync_copy(z2_v, acc_sh.at[pl.ds(r0, ZR), :])
            return _

        lax.fori_loop(0, zs // ZR, zbody, None)
        for j in range(CH // 16):
            one_v[pl.ds(j * 16, 16)] = jnp.ones((16,), jnp.float32)
        plsc.subcore_barrier()

        base = jnp.int32(c * NHALF)

        def body(kk, _):
            off = (s * CPS + kk) * CH
            pltpu.sync_copy(tgt_hbm.at[pl.ds(off, CH)], ti_v)
            pltpu.sync_copy(rows_hbm.at[pl.ds(off, CH), :], rv_v)
            for j in range(CH // 16):
                iv = ti_v[pl.ds(j * 16, 16)] - base
                ok = (iv >= 0) & (iv < NHALF)
                ix_v[pl.ds(j * 16, 16)] = jnp.where(ok, iv, GARB)
            pltpu.sync_copy(rv_v, acc_sh.at[ix_v], add=True)
            pltpu.sync_copy(one_v, cnt_sh.at[ix_v], add=True)
            return _

        lax.fori_loop(0, CPS, body, None)
        plsc.subcore_barrier()

        pltpu.sync_copy(cnt_sh.at[pl.ds(s * SR, SR)], cb_v)
        pltpu.sync_copy(cb_v, cnt_out.at[c, pl.ds(s * SR, SR)])

        def wbody(kk, _):
            r0 = s * SR + kk * 56
            pltpu.sync_copy(acc_sh.at[pl.ds(r0, 56), :], rb_v)
            pltpu.sync_copy(rb_v, sum_out.at[c, pl.ds(r0, 56), :])
            return _

        lax.fori_loop(0, SR // 56, wbody, None)

    return k(tgt_m, rows, z1d, z2d)


def _mm(a, b):
    return jax.lax.dot_general(a, b, (((1,), (0,)), ((), ())),
                               preferred_element_type=jnp.float32,
                               precision=jax.lax.Precision.HIGHEST)


def _mmb(a, b):
    return jax.lax.dot_general(a.astype(jnp.bfloat16), b.astype(jnp.bfloat16),
                               (((1,), (0,)), ((), ())),
                               preferred_element_type=jnp.float32)


def _b32(x):
    return x.astype(jnp.bfloat16).astype(jnp.float32)


def _mms(a, b):
    hi = _b32(a)
    lo = a - hi
    return _mmb(hi, b) + _mmb(lo, b)


def _tc_pass1(hs, ht, ps, pt, B1a, B1b, B1c, S, E):
    EP = hs.shape[0]
    grid = EP // BLK

    def body(hs_r, ht_r, ps_r, pt_r, b1a_r, b1b_r, b1c_r, s_r,
             x1_r, dist_r, st_r):
        i = pl.program_id(0)
        r8 = pt_r[...] - ps_r[...]
        lane = lax.broadcasted_iota(jnp.int32, (BLK, 8), 1)
        r8 = jnp.where(lane < 3, r8, 0.0)
        d2 = jnp.sum(r8 * r8, axis=1, keepdims=True)
        dist = jnp.sqrt(d2)
        dist_r[...] = dist
        x1 = (_mmb(hs_r[...], b1a_r[...]) + _mmb(ht_r[...], b1b_r[...])
              + _mmb(r8, b1c_r[...]))
        x1_r[...] = x1
        n1 = jnp.sqrt(_mms(x1 * x1, s_r[...]))
        row = i * BLK + lax.broadcasted_iota(jnp.int32, (BLK, 16), 0)
        n1 = jnp.where(row < E, n1, 0.0)
        st = jnp.stack([jnp.sum(n1, axis=0), jnp.sum(n1 * n1, axis=0)])

        @pl.when(i == 0)
        def _():
            st_r[...] = jnp.zeros_like(st_r)

        st_r[...] += st

    return pl.pallas_call(
        body,
        grid=(grid,),
        in_specs=[
            pl.BlockSpec((BLK, 48), lambda i: (i, 0)),
            pl.BlockSpec((BLK, 48), lambda i: (i, 0)),
            pl.BlockSpec((BLK, 8), lambda i: (i, 0)),
            pl.BlockSpec((BLK, 8), lambda i: (i, 0)),
            pl.BlockSpec((48, 48), lambda i: (0, 0)),
            pl.BlockSpec((48, 48), lambda i: (0, 0)),
            pl.BlockSpec((8, 48), lambda i: (0, 0)),
            pl.BlockSpec((48, 16), lambda i: (0, 0)),
        ],
        out_specs=[
            pl.BlockSpec((BLK, 48), lambda i: (i, 0)),
            pl.BlockSpec((BLK, 1), lambda i: (i, 0)),
            pl.BlockSpec((2, 16), lambda i: (0, 0)),
        ],
        out_shape=[
            jax.ShapeDtypeStruct((EP, 48), jnp.float32),
            jax.ShapeDtypeStruct((EP, 1), jnp.float32),
            jax.ShapeDtypeStruct((2, 16), jnp.float32),
        ],
    )(hs, ht, ps, pt, B1a, B1b, B1c, S)


def _bn_lrelu(x, S, ST, mean, std, gamma, dhat):
    nsq = _mms(x * x, S)
    n = jnp.sqrt(nsq)
    f = (n - mean) / ((n + EPS_BN) * std) * gamma
    xbn = x * _mms(f, ST)
    proj = _mms(xbn * dhat, S)
    fac = SLOPE + (1.0 - SLOPE) * (proj >= 0).astype(jnp.float32)
    return xbn * _mmb(fac, ST)


def _tc_pass2(x1, B2, S, ST, mean1, std1, gamma1, dhat1, E):
    EP = x1.shape[0]
    grid = EP // BLK

    def body(x1_r, b2_r, s_r, st_rr, m_r, sd_r, g_r, d_r, x2_r, st_r):
        i = pl.program_id(0)
        y1 = _bn_lrelu(x1_r[...], s_r[...], st_rr[...],
                       m_r[...], sd_r[...], g_r[...], d_r[...])
        x2 = _mmb(y1, b2_r[...])
        x2_r[...] = x2
        n2 = jnp.sqrt(_mms(x2 * x2, s_r[...]))
        row = i * BLK + lax.broadcasted_iota(jnp.int32, (BLK, 16), 0)
        n2 = jnp.where(row < E, n2, 0.0)
        st = jnp.stack([jnp.sum(n2, axis=0), jnp.sum(n2 * n2, axis=0)])

        @pl.when(i == 0)
        def _():
            st_r[...] = jnp.zeros_like(st_r)

        st_r[...] += st

    return pl.pallas_call(
        body,
        grid=(grid,),
        in_specs=[
            pl.BlockSpec((BLK, 48), lambda i: (i, 0)),
            pl.BlockSpec((48, 48), lambda i: (0, 0)),
            pl.BlockSpec((48, 16), lambda i: (0, 0)),
            pl.BlockSpec((16, 48), lambda i: (0, 0)),
            pl.BlockSpec((1, 16), lambda i: (0, 0)),
            pl.BlockSpec((1, 16), lambda i: (0, 0)),
            pl.BlockSpec((1, 16), lambda i: (0, 0)),
            pl.BlockSpec((1, 48), lambda i: (0, 0)),
        ],
        out_specs=[
            pl.BlockSpec((BLK, 48), lambda i: (i, 0)),
            pl.BlockSpec((2, 16), lambda i: (0, 0)),
        ],
        out_shape=[
            jax.ShapeDtypeStruct((EP, 48), jnp.float32),
            jax.ShapeDtypeStruct((2, 16), jnp.float32),
        ],
    )(x1, B2, S, ST, mean1, std1, gamma1, dhat1)


def _tc_pass3(x2, dist, ld, S, ST, mean2, std2, gamma2, dhat2,
              W1T, b1, W2T, b2, WmT, bm):
    EP = x2.shape[0]
    grid = EP // BLK

    def body(x2_r, dist_r, ld_r, s_r, st_rr, m_r, sd_r, g_r, d_r,
             w1_r, b1_r, w2_r, b2_r, wm_r, bm_r, mf_r):
        y2 = _bn_lrelu(x2_r[...], s_r[...], st_rr[...],
                       m_r[...], sd_r[...], g_r[...], d_r[...])
        dist = dist_r[...]
        ld = ld_r[...]
        w1 = _b32(w1_r[...])
        zp = (_b32(dist) * w1[0:1, :] + _b32(dist / (ld + 1e-6)) * w1[1:2, :]
              + _b32(jnp.log(dist + 1e-6)) * w1[2:3, :] + _b32(ld) * w1[3:4, :]
              + b1_r[...])
        z = zp * jax.nn.sigmoid(zp)
        ie = _mmb(z, w2_r[...]) + b2_r[...]
        mod = jax.nn.sigmoid(_mmb(ie, wm_r[...]) + bm_r[...])
        mf_r[...] = y2 * _mmb(mod, st_rr[...])

    return pl.pallas_call(
        body,
        grid=(grid,),
        in_specs=[
            pl.BlockSpec((BLK, 48), lambda i: (i, 0)),
            pl.BlockSpec((BLK, 1), lambda i: (i, 0)),
            pl.BlockSpec((BLK, 1), lambda i: (i, 0)),
            pl.BlockSpec((48, 16), lambda i: (0, 0)),
            pl.BlockSpec((16, 48), lambda i: (0, 0)),
            pl.BlockSpec((1, 16), lambda i: (0, 0)),
            pl.BlockSpec((1, 16), lambda i: (0, 0)),
            pl.BlockSpec((1, 16), lambda i: (0, 0)),
            pl.BlockSpec((1, 48), lambda i: (0, 0)),
            pl.BlockSpec((4, 16), lambda i: (0, 0)),
            pl.BlockSpec((1, 16), lambda i: (0, 0)),
            pl.BlockSpec((16, 16), lambda i: (0, 0)),
            pl.BlockSpec((1, 16), lambda i: (0, 0)),
            pl.BlockSpec((16, 16), lambda i: (0, 0)),
            pl.BlockSpec((1, 16), lambda i: (0, 0)),
        ],
        out_specs=pl.BlockSpec((BLK, 48), lambda i: (i, 0)),
        out_shape=jax.ShapeDtypeStruct((EP, 48), jnp.float32),
    )(x2, dist, ld, S, ST, mean2, std2, gamma2, dhat2,
      W1T, b1, W2T, b2, WmT, bm)


def _tc_divide(sums, cnt):
    NP = sums.shape[0]
    blk = 1024
    grid = NP // blk

    def body(s_r, c_r, o_r):
        o_r[...] = s_r[...] / jnp.maximum(c_r[...], 1.0)

    return pl.pallas_call(
        body,
        grid=(grid,),
        in_specs=[
            pl.BlockSpec((blk, 48), lambda i: (i, 0)),
            pl.BlockSpec((blk, 1), lambda i: (i, 0)),
        ],
        out_specs=pl.BlockSpec((blk, 48), lambda i: (i, 0)),
        out_shape=jax.ShapeDtypeStruct((NP, 48), jnp.float32),
    )(sums, cnt)


def kernel(h, edge_index, pos, batch, inv_W1, inv_b1, inv_W2, inv_b2,
           mod_W, mod_b, lin1_W, bn1_gamma, dir1, lin2_W, bn2_gamma, dir2):
    N = h.shape[0]
    E = edge_index.shape[1]
    C = h.shape[1]

    EP = -(-E // (CH * NW)) * (CH * NW)
    NHALF = -(-N // 2)
    SR = -(-NHALF // (NS * 16)) * 16
    NH_OUT = NS * SR
    NACC = -(-(NH_OUT + 16) // (NS * 64)) * NS * 64

    src = edge_index[0].astype(jnp.int32)
    tgt = edge_index[1].astype(jnp.int32)
    pad = EP - E
    src0 = jnp.pad(src, (0, pad))
    tgt0 = jnp.pad(tgt, (0, pad))
    srcm = jnp.pad(src, (0, pad), constant_values=-1)
    tgtm = jnp.pad(tgt, (0, pad), constant_values=-1)
    h48 = h.reshape(N, 3 * C)
    pos8 = jnp.pad(pos, ((0, 0), (0, 5)))

    eye3 = jnp.eye(3, dtype=jnp.float32)
    B1a = jnp.kron(lin1_W[:, 0:C].T, eye3)
    B1b = jnp.kron(lin1_W[:, C:2 * C].T, eye3)
    B1c = jnp.kron(lin1_W[:, 2 * C:].T, eye3)
    B1c = jnp.pad(B1c, ((0, 8 - B1c.shape[0]), (0, 0)))
    B2 = jnp.kron(lin2_W.T, eye3)
    S = jnp.kron(jnp.eye(C, dtype=jnp.float32), jnp.ones((3, 1), jnp.float32))
    ST = S.T

    def _dhat(d):
        dd = d / jnp.clip(jnp.linalg.norm(d, axis=-1, keepdims=True),
                          1e-12, None)
        return dd.reshape(1, 3 * C)

    dhat1 = _dhat(dir1)
    dhat2 = _dhat(dir2)
    g1 = bn1_gamma.reshape(1, C)
    g2 = bn2_gamma.reshape(1, C)

    z1d = jnp.zeros((NACC // NS,), jnp.float32)
    z2d = jnp.zeros((64, 48), jnp.float32)

    hs, ht, ps, pt = _sc_gather4(h48, pos8, src0, tgt0, EP)
    x1, dist, st1 = _tc_pass1(hs, ht, ps, pt, B1a, B1b, B1c, S, E)

    nmd2 = _sc_scatter_mean1(srcm, dist.reshape(EP), EP, NHALF, SR, NACC, z1d)
    nmd = jnp.concatenate([nmd2[0, :NHALF], nmd2[1, :N - NHALF]])
    nmd8 = jnp.tile(nmd.reshape(N, 1), (1, 8))
    ld = _sc_gather1(nmd8, src0, EP)[:, :1]

    def _finalize(st):
        mean = st[0] / E
        var = (st[1] - E * mean * mean) / (E - 1)
        std = jnp.sqrt(jnp.maximum(var, 0.0)) + EPS_BN
        return mean.reshape(1, C), std.reshape(1, C)

    mean1, std1 = _finalize(st1)
    x2, st2 = _tc_pass2(x1, B2, S, ST, mean1, std1, g1, dhat1, E)
    mean2, std2 = _finalize(st2)

    mf = _tc_pass3(x2, dist, ld, S, ST, mean2, std2, g2, dhat2,
                   inv_W1.T, inv_b1.reshape(1, C), inv_W2.T,
                   inv_b2.reshape(1, C), mod_W.T, mod_b.reshape(1, C))

    msum, mcnt = _sc_scatter_rows(tgtm, mf, EP, NHALF, SR, NACC, z1d, z2d)
    sums = jnp.concatenate([msum[0, :NHALF], msum[1, :N - NHALF]], axis=0)
    cnts = jnp.concatenate([mcnt[0, :NHALF], mcnt[1, :N - NHALF]])
    NP6 = -(-N // 1024) * 1024
    sums = jnp.pad(sums, ((0, NP6 - N), (0, 0)))
    cnts = jnp.pad(cnts, (0, NP6 - N)).reshape(NP6, 1)
    out = _tc_divide(sums, cnts)[:N]
    return out.reshape(N, C, 3)
